# Initial kernel scaffold; baseline (speedup 1.0000x reference)
import functools

import jax
import jax.numpy as jnp
from jax import lax
from jax.experimental import pallas as pl
from jax.experimental.pallas import tpu as pltpu
from jax.experimental.pallas import tpu_sc as plsc

E = 8
TOPK = 2
T = 2048
D = 1024
FF = 3584

BT = 256
NBMAX = 16 + E - 1
P = NBMAX * BT
FFB = 512
NF = FF // FFB

NTILES = 32
TPT = T // NTILES
LANES = 128


def _routing_body(x_ref, gw_ref, pos0_ref, pos1_ref, w0_ref, w1_ref, meta_ref):
    x = x_ref[...]
    gw = gw_ref[...]
    logits = lax.dot_general(x, gw, (((1,), (1,)), ((), ())),
                             preferred_element_type=jnp.float32)
    lane = lax.broadcasted_iota(jnp.int32, (T, LANES), 1)
    neg = jnp.float32(-1e30)
    logits = jnp.where(lane < E, logits, neg)

    m0 = jnp.max(logits, axis=1, keepdims=True)
    i0 = jnp.min(jnp.where(logits == m0, lane, LANES), axis=1, keepdims=True)
    l2 = jnp.where(lane == i0, neg, logits)
    m1 = jnp.max(l2, axis=1, keepdims=True)
    i1 = jnp.min(jnp.where(l2 == m1, lane, LANES), axis=1, keepdims=True)

    ex = jnp.exp(m1 - m0)
    w0 = 1.0 / (1.0 + ex)
    w1 = ex / (1.0 + ex)

    oh0 = (lane == i0).astype(jnp.float32)
    oh1 = (lane == i1).astype(jnp.float32)
    cnt = oh0 + oh1

    csum = cnt
    s = 1
    while s < T:
        csum = csum + jnp.concatenate(
            [jnp.zeros((s, LANES), jnp.float32), csum[:-s, :]], axis=0)
        s *= 2
    excl = csum - cnt
    counts = csum[T - 1:T, :]

    nb = jnp.floor((counts + (BT - 1)) / BT)
    nb = jnp.where(lane[:1, :] < E, nb, 0.0)
    pnb = nb
    s = 1
    while s < LANES:
        pnb = pnb + jnp.concatenate(
            [jnp.zeros((1, s), jnp.float32), pnb[:, :-s]], axis=1)
        s *= 2
    pext = pnb - nb
    padded_off = BT * pext

    slot = excl + padded_off
    pos0 = jnp.sum(oh0 * slot, axis=1, keepdims=True)
    pos1 = jnp.sum(oh1 * slot, axis=1, keepdims=True)

    nact = jnp.sum(jnp.where(lane[:1, :] == E - 1, pnb, 0.0),
                   axis=1, keepdims=True)
    bf = lane[:1, :].astype(jnp.float32)
    be = jnp.zeros((1, LANES), jnp.float32)
    for e in range(E):
        pnb_e = jnp.sum(jnp.where(lane[:1, :] == e, pnb, 0.0),
                        axis=1, keepdims=True)
        be = be + (bf >= pnb_e).astype(jnp.float32)
    be = jnp.minimum(be, float(E - 1))

    meta = jnp.where(lane[:1, :] < NBMAX, be,
                     jnp.where(lane[:1, :] == NBMAX, nact, 0.0))

    pos0_ref[...] = pos0.astype(jnp.int32)
    pos1_ref[...] = pos1.astype(jnp.int32)
    w0_ref[...] = w0
    w1_ref[...] = w1
    meta_ref[...] = meta.astype(jnp.int32)


def _routing(x, gw_pad):
    return pl.pallas_call(
        _routing_body,
        out_shape=[
            jax.ShapeDtypeStruct((T, 1), jnp.int32),
            jax.ShapeDtypeStruct((T, 1), jnp.int32),
            jax.ShapeDtypeStruct((T, 1), jnp.float32),
            jax.ShapeDtypeStruct((T, 1), jnp.float32),
            jax.ShapeDtypeStruct((1, LANES), jnp.int32),
        ],
    )(x, gw_pad)


def _dispatch_body(x_hbm, p0_hbm, p1_hbm, out_hbm, idx0_v, idx1_v, rows_v, sem):
    c = lax.axis_index("c")
    s = lax.axis_index("s")
    wid = s * 2 + c
    pltpu.sync_copy(p0_hbm.at[wid], idx0_v)
    pltpu.sync_copy(p1_hbm.at[wid], idx1_v)
    pltpu.sync_copy(x_hbm.at[pl.ds(wid * TPT, TPT)], rows_v)
    pltpu.async_copy(rows_v, out_hbm.at[idx0_v], sem).wait()
    pltpu.async_copy(rows_v, out_hbm.at[idx1_v], sem).wait()


def _dispatch(x, p0, p1):
    mesh = plsc.VectorSubcoreMesh(core_axis_name="c", subcore_axis_name="s")
    fn = functools.partial(
        pl.kernel,
        out_type=jax.ShapeDtypeStruct((P, D), jnp.float32),
        mesh=mesh,
        scratch_types=[
            pltpu.VMEM((TPT,), jnp.int32),
            pltpu.VMEM((TPT,), jnp.int32),
            pltpu.VMEM((TPT, D), jnp.float32),
            pltpu.SemaphoreType.DMA,
        ],
    )(_dispatch_body)
    return fn(x, p0, p1)


def _gmm_body(bex_ref, nact_ref, xs_ref, w1_ref, w3_ref, w2_ref, out_ref):
    b = pl.program_id(0)
    f = pl.program_id(1)

    @pl.when(b < nact_ref[0])
    def _():
        x = xs_ref[...]
        h1 = lax.dot_general(x, w1_ref[0], (((1,), (1,)), ((), ())),
                             preferred_element_type=jnp.float32)
        h3 = lax.dot_general(x, w3_ref[0], (((1,), (1,)), ((), ())),
                             preferred_element_type=jnp.float32)
        h = h1 * lax.logistic(h1) * h3
        y = lax.dot_general(h, w2_ref[0], (((1,), (1,)), ((), ())),
                            preferred_element_type=jnp.float32)

        @pl.when(f == 0)
        def _():
            out_ref[...] = y

        @pl.when(f > 0)
        def _():
            out_ref[...] += y


def _gmm(bex, nact, xs, w1, w3, w2):
    def expert_of(b, bex_ref, nact_ref):
        return bex_ref[jnp.minimum(b, nact_ref[0] - 1)]

    grid_spec = pltpu.PrefetchScalarGridSpec(
        num_scalar_prefetch=2,
        grid=(NBMAX, NF),
        in_specs=[
            pl.BlockSpec((BT, D), lambda b, f, bex, nact: (b, 0)),
            pl.BlockSpec((1, FFB, D),
                         lambda b, f, bex, nact: (expert_of(b, bex, nact), f, 0)),
            pl.BlockSpec((1, FFB, D),
                         lambda b, f, bex, nact: (expert_of(b, bex, nact), f, 0)),
            pl.BlockSpec((1, D, FFB),
                         lambda b, f, bex, nact: (expert_of(b, bex, nact), 0, f)),
        ],
        out_specs=pl.BlockSpec((BT, D), lambda b, f, bex, nact: (b, 0)),
    )
    return pl.pallas_call(
        _gmm_body,
        grid_spec=grid_spec,
        out_shape=jax.ShapeDtypeStruct((P, D), jnp.float32),
        compiler_params=pltpu.CompilerParams(
            dimension_semantics=("arbitrary", "arbitrary")),
    )(bex, nact, xs, w1, w3, w2)


def _combine_body(ys_hbm, p0_hbm, p1_hbm, w0_hbm, w1_hbm, out_hbm,
                  idx0_v, idx1_v, w0_v, w1_v, g0_v, g1_v, sem0, sem1):
    c = lax.axis_index("c")
    s = lax.axis_index("s")
    wid = s * 2 + c
    pltpu.sync_copy(p0_hbm.at[wid], idx0_v)
    pltpu.sync_copy(p1_hbm.at[wid], idx1_v)
    pltpu.sync_copy(w0_hbm.at[wid], w0_v)
    pltpu.sync_copy(w1_hbm.at[wid], w1_v)
    half_n = TPT // 2
    for half in range(2):
        cp0 = pltpu.async_copy(
            ys_hbm.at[idx0_v.at[pl.ds(half * half_n, half_n)]], g0_v, sem0)
        cp1 = pltpu.async_copy(
            ys_hbm.at[idx1_v.at[pl.ds(half * half_n, half_n)]], g1_v, sem1)
        cp0.wait()
        cp1.wait()

        def row_body(r, _, half=half):
            ridx = jnp.full((16,), half * half_n + r, jnp.int32)
            a = plsc.load_gather(w0_v, [ridx])
            bw = plsc.load_gather(w1_v, [ridx])
            for cc in range(D // 16):
                sl = pl.ds(cc * 16, 16)
                g0_v[r, sl] = a * g0_v[r, sl] + bw * g1_v[r, sl]
            return 0

        lax.fori_loop(0, half_n, row_body, 0)
        pltpu.sync_copy(g0_v, out_hbm.at[pl.ds(wid * TPT + half * half_n, half_n)])


def _combine(ys, p0, p1, w0m, w1m):
    mesh = plsc.VectorSubcoreMesh(core_axis_name="c", subcore_axis_name="s")
    half_n = TPT // 2
    fn = functools.partial(
        pl.kernel,
        out_type=jax.ShapeDtypeStruct((T, D), jnp.float32),
        mesh=mesh,
        scratch_types=[
            pltpu.VMEM((TPT,), jnp.int32),
            pltpu.VMEM((TPT,), jnp.int32),
            pltpu.VMEM((TPT,), jnp.float32),
            pltpu.VMEM((TPT,), jnp.float32),
            pltpu.VMEM((half_n, D), jnp.float32),
            pltpu.VMEM((half_n, D), jnp.float32),
            pltpu.SemaphoreType.DMA,
            pltpu.SemaphoreType.DMA,
        ],
    )(_combine_body)
    return fn(ys, p0, p1, w0m, w1m)


def kernel(hidden_states, gate_w, w1, w2, w3):
    b, s, d = hidden_states.shape
    x = hidden_states.reshape(-1, d)
    gw_pad = jnp.pad(gate_w, ((0, LANES - E), (0, 0)))
    pos0, pos1, w0c, w1c, meta = _routing(x, gw_pad)
    p0 = pos0.reshape(NTILES, TPT)
    p1 = pos1.reshape(NTILES, TPT)
    w0m = w0c.reshape(NTILES, TPT)
    w1m = w1c.reshape(NTILES, TPT)
    bex = meta[0, :NBMAX]
    nact = meta[0, NBMAX:NBMAX + 1]
    xs = _dispatch(x, p0, p1)
    ys = _gmm(bex, nact, xs, w1, w3, w2)
    out = _combine(ys, p0, p1, w0m, w1m)
    return out.reshape(b, s, d)


# speedup vs baseline: 1.5201x; 1.5201x over previous
import functools

import jax
import jax.numpy as jnp
from jax import lax
from jax.experimental import pallas as pl
from jax.experimental.pallas import tpu as pltpu
from jax.experimental.pallas import tpu_sc as plsc

E = 8
TOPK = 2
T = 2048
D = 1024
FF = 3584

BT = 256
NBMAX = 16 + E - 1
P = NBMAX * BT
FFB = 512
NF = FF // FFB

NTILES = 32
TPT = T // NTILES
LANES = 128


def _routing_body(x_ref, gw_ref, pos0_ref, pos1_ref, w0_ref, w1_ref, meta_ref):
    x = x_ref[...]
    gw = gw_ref[...]
    logits = lax.dot_general(x, gw, (((1,), (1,)), ((), ())),
                             preferred_element_type=jnp.float32)
    lane = lax.broadcasted_iota(jnp.int32, (T, LANES), 1)
    neg = jnp.float32(-1e30)
    logits = jnp.where(lane < E, logits, neg)

    m0 = jnp.max(logits, axis=1, keepdims=True)
    i0 = jnp.min(jnp.where(logits == m0, lane, LANES), axis=1, keepdims=True)
    l2 = jnp.where(lane == i0, neg, logits)
    m1 = jnp.max(l2, axis=1, keepdims=True)
    i1 = jnp.min(jnp.where(l2 == m1, lane, LANES), axis=1, keepdims=True)

    ex = jnp.exp(m1 - m0)
    w0 = 1.0 / (1.0 + ex)
    w1 = ex / (1.0 + ex)

    oh0 = (lane == i0).astype(jnp.float32)
    oh1 = (lane == i1).astype(jnp.float32)
    cnt = oh0 + oh1

    csum = cnt
    s = 1
    while s < T:
        csum = csum + jnp.concatenate(
            [jnp.zeros((s, LANES), jnp.float32), csum[:-s, :]], axis=0)
        s *= 2
    excl = csum - cnt
    counts = csum[T - 1:T, :]

    nb = jnp.floor((counts + (BT - 1)) / BT)
    nb = jnp.where(lane[:1, :] < E, nb, 0.0)
    pnb = nb
    s = 1
    while s < LANES:
        pnb = pnb + jnp.concatenate(
            [jnp.zeros((1, s), jnp.float32), pnb[:, :-s]], axis=1)
        s *= 2
    pext = pnb - nb
    padded_off = BT * pext

    slot = excl + padded_off
    pos0 = jnp.sum(oh0 * slot, axis=1, keepdims=True)
    pos1 = jnp.sum(oh1 * slot, axis=1, keepdims=True)

    nact = jnp.sum(jnp.where(lane[:1, :] == E - 1, pnb, 0.0),
                   axis=1, keepdims=True)
    bf = lane[:1, :].astype(jnp.float32)
    be = jnp.zeros((1, LANES), jnp.float32)
    for e in range(E):
        pnb_e = jnp.sum(jnp.where(lane[:1, :] == e, pnb, 0.0),
                        axis=1, keepdims=True)
        be = be + (bf >= pnb_e).astype(jnp.float32)
    be = jnp.minimum(be, float(E - 1))

    meta = jnp.where(lane[:1, :] < NBMAX, be,
                     jnp.where(lane[:1, :] == NBMAX, nact, 0.0))

    pos0_ref[...] = pos0.astype(jnp.int32)
    pos1_ref[...] = pos1.astype(jnp.int32)
    w0_ref[...] = jnp.broadcast_to(w0, (T, 16))
    w1_ref[...] = jnp.broadcast_to(w1, (T, 16))
    meta_ref[...] = meta.astype(jnp.int32)


def _routing(x, gw_pad):
    return pl.pallas_call(
        _routing_body,
        out_shape=[
            jax.ShapeDtypeStruct((T, 1), jnp.int32),
            jax.ShapeDtypeStruct((T, 1), jnp.int32),
            jax.ShapeDtypeStruct((T, 16), jnp.float32),
            jax.ShapeDtypeStruct((T, 16), jnp.float32),
            jax.ShapeDtypeStruct((1, LANES), jnp.int32),
        ],
    )(x, gw_pad)


def _dispatch_body(x_hbm, p0_hbm, p1_hbm, out_hbm, idx0_v, idx1_v, rows_v, sem):
    c = lax.axis_index("c")
    s = lax.axis_index("s")
    wid = s * 2 + c
    pltpu.sync_copy(p0_hbm.at[wid], idx0_v)
    pltpu.sync_copy(p1_hbm.at[wid], idx1_v)
    pltpu.sync_copy(x_hbm.at[pl.ds(wid * TPT, TPT)], rows_v)
    pltpu.async_copy(rows_v, out_hbm.at[idx0_v], sem).wait()
    pltpu.async_copy(rows_v, out_hbm.at[idx1_v], sem).wait()


def _dispatch(x, p0, p1):
    mesh = plsc.VectorSubcoreMesh(core_axis_name="c", subcore_axis_name="s")
    fn = functools.partial(
        pl.kernel,
        out_type=jax.ShapeDtypeStruct((P, D), jnp.float32),
        mesh=mesh,
        scratch_types=[
            pltpu.VMEM((TPT,), jnp.int32),
            pltpu.VMEM((TPT,), jnp.int32),
            pltpu.VMEM((TPT, D), jnp.float32),
            pltpu.SemaphoreType.DMA,
        ],
    )(_dispatch_body)
    return fn(x, p0, p1)


def _gmm_body(bex_ref, nact_ref, xs_ref, w1_ref, w3_ref, w2_ref, out_ref):
    b = pl.program_id(0)
    f = pl.program_id(1)

    @pl.when(b < nact_ref[0])
    def _():
        x = xs_ref[...]
        h1 = lax.dot_general(x, w1_ref[0], (((1,), (1,)), ((), ())),
                             preferred_element_type=jnp.float32)
        h3 = lax.dot_general(x, w3_ref[0], (((1,), (1,)), ((), ())),
                             preferred_element_type=jnp.float32)
        h = h1 * lax.logistic(h1) * h3
        y = lax.dot_general(h, w2_ref[0], (((1,), (1,)), ((), ())),
                            preferred_element_type=jnp.float32)

        @pl.when(f == 0)
        def _():
            out_ref[...] = y

        @pl.when(f > 0)
        def _():
            out_ref[...] += y


def _gmm(bex, nact, xs, w1, w3, w2):
    def expert_of(b, bex_ref, nact_ref):
        return bex_ref[jnp.minimum(b, nact_ref[0] - 1)]

    grid_spec = pltpu.PrefetchScalarGridSpec(
        num_scalar_prefetch=2,
        grid=(NBMAX, NF),
        in_specs=[
            pl.BlockSpec((BT, D), lambda b, f, bex, nact: (b, 0)),
            pl.BlockSpec((1, FFB, D),
                         lambda b, f, bex, nact: (expert_of(b, bex, nact), f, 0)),
            pl.BlockSpec((1, FFB, D),
                         lambda b, f, bex, nact: (expert_of(b, bex, nact), f, 0)),
            pl.BlockSpec((1, D, FFB),
                         lambda b, f, bex, nact: (expert_of(b, bex, nact), 0, f)),
        ],
        out_specs=pl.BlockSpec((BT, D), lambda b, f, bex, nact: (b, 0)),
    )
    return pl.pallas_call(
        _gmm_body,
        grid_spec=grid_spec,
        out_shape=jax.ShapeDtypeStruct((P, D), jnp.float32),
        compiler_params=pltpu.CompilerParams(
            dimension_semantics=("arbitrary", "arbitrary")),
    )(bex, nact, xs, w1, w3, w2)


def _combine_body(ys_hbm, p0_hbm, p1_hbm, w0_hbm, w1_hbm, out_hbm,
                  idx0_v, idx1_v, w0_v, w1_v, g0_v, g1_v, sem0, sem1):
    c = lax.axis_index("c")
    s = lax.axis_index("s")
    wid = s * 2 + c
    pltpu.sync_copy(p0_hbm.at[wid], idx0_v)
    pltpu.sync_copy(p1_hbm.at[wid], idx1_v)
    pltpu.sync_copy(w0_hbm.at[wid], w0_v)
    pltpu.sync_copy(w1_hbm.at[wid], w1_v)
    half_n = TPT // 2
    for half in range(2):
        cp0 = pltpu.async_copy(
            ys_hbm.at[idx0_v.at[pl.ds(half * half_n, half_n)]], g0_v, sem0)
        cp1 = pltpu.async_copy(
            ys_hbm.at[idx1_v.at[pl.ds(half * half_n, half_n)]], g1_v, sem1)
        cp0.wait()
        cp1.wait()

        def row_body(r, _, half=half):
            a = w0_v[half * half_n + r, :]
            bw = w1_v[half * half_n + r, :]
            for cc in range(D // 16):
                sl = pl.ds(cc * 16, 16)
                g0_v[r, sl] = a * g0_v[r, sl] + bw * g1_v[r, sl]
            return 0

        lax.fori_loop(0, half_n, row_body, 0)
        pltpu.sync_copy(g0_v, out_hbm.at[pl.ds(wid * TPT + half * half_n, half_n)])


def _combine(ys, p0, p1, w0m, w1m):
    mesh = plsc.VectorSubcoreMesh(core_axis_name="c", subcore_axis_name="s")
    half_n = TPT // 2
    fn = functools.partial(
        pl.kernel,
        out_type=jax.ShapeDtypeStruct((T, D), jnp.float32),
        mesh=mesh,
        scratch_types=[
            pltpu.VMEM((TPT,), jnp.int32),
            pltpu.VMEM((TPT,), jnp.int32),
            pltpu.VMEM((TPT, 16), jnp.float32),
            pltpu.VMEM((TPT, 16), jnp.float32),
            pltpu.VMEM((half_n, D), jnp.float32),
            pltpu.VMEM((half_n, D), jnp.float32),
            pltpu.SemaphoreType.DMA,
            pltpu.SemaphoreType.DMA,
        ],
    )(_combine_body)
    return fn(ys, p0, p1, w0m, w1m)


def kernel(hidden_states, gate_w, w1, w2, w3):
    b, s, d = hidden_states.shape
    x = hidden_states.reshape(-1, d)
    gw_pad = jnp.pad(gate_w, ((0, LANES - E), (0, 0)))
    pos0, pos1, w0c, w1c, meta = _routing(x, gw_pad)
    p0 = pos0.reshape(NTILES, TPT)
    p1 = pos1.reshape(NTILES, TPT)
    w0m = w0c.reshape(NTILES, TPT, 16)
    w1m = w1c.reshape(NTILES, TPT, 16)
    bex = meta[0, :NBMAX]
    nact = meta[0, NBMAX:NBMAX + 1]
    xs = _dispatch(x, p0, p1)
    ys = _gmm(bex, nact, xs, w1, w3, w2)
    out = _combine(ys, p0, p1, w0m, w1m)
    return out.reshape(b, s, d)
